# baseline (speedup 1.0000x reference)
import jax
import jax.numpy as jnp
from jax import lax
from jax.experimental import pallas as pl
from jax.experimental.pallas import tpu as pltpu
from jax.experimental.pallas import tpu_sc as plsc

_B, _M, _F, _H = 16, 4096, 64, 16


_BI = 2
_NSTEP = _B // _BI


def _subnet_body(x_hbm, w1t_ref, b1r_ref, w2t_ref, b2_ref, out_ref,
                 xbuf, sems):
    i = pl.program_id(0)
    nbuf = 3

    @pl.when(i == 0)
    def _prime():
        for k in range(nbuf - 1):
            pltpu.make_async_copy(x_hbm.at[pl.ds(k * _BI, _BI)],
                                  xbuf.at[k], sems.at[k]).start()

    @pl.when(i + nbuf - 1 < _NSTEP)
    def _next():
        j = i + nbuf - 1
        pltpu.make_async_copy(x_hbm.at[pl.ds(j * _BI, _BI)],
                              xbuf.at[j % nbuf], sems.at[j % nbuf]).start()

    pltpu.make_async_copy(x_hbm.at[pl.ds(i * _BI, _BI)],
                          xbuf.at[i % nbuf], sems.at[i % nbuf]).wait()
    b1c = b1r_ref[...].reshape(_H, 1)
    for b in range(_BI):
        x = xbuf[i % nbuf, b]
        zt = lax.dot_general(w1t_ref[...], x, (((1,), (0,)), ((), ())),
                             preferred_element_type=jnp.float32)
        ht = jnp.maximum(zt + b1c, 0.0)
        c = lax.dot_general(w2t_ref[...], ht, (((1,), (0,)), ((), ())),
                            preferred_element_type=jnp.float32)
        out_ref[pl.ds(b * _M, _M)] = (c + b2_ref[0, 0]).reshape(_M)


def _subnet_contribs(xt, w1t, b1r, w2t, b2r):
    return pl.pallas_call(
        _subnet_body,
        grid=(_NSTEP,),
        in_specs=[
            pl.BlockSpec(memory_space=pl.ANY),
            pl.BlockSpec((_H, _F), lambda i: (0, 0)),
            pl.BlockSpec((1, _H), lambda i: (0, 0)),
            pl.BlockSpec((1, _H), lambda i: (0, 0)),
            pl.BlockSpec((1, 1), lambda i: (0, 0)),
        ],
        out_specs=pl.BlockSpec((_BI * _M,), lambda i: (i,)),
        out_shape=jax.ShapeDtypeStruct((_B * _M,), jnp.float32),
        scratch_shapes=[
            pltpu.VMEM((3, _BI, _F, _M), jnp.float32),
            pltpu.SemaphoreType.DMA((3,)),
        ],
    )(xt, w1t, b1r, w2t, b2r)


def _lane_shuffle(v, idx):
    return lax.gather(
        v, idx[:, None],
        lax.GatherDimensionNumbers(offset_dims=(), collapsed_slice_dims=(0,),
                                   start_index_map=(0,)),
        slice_sizes=(1,), mode=lax.GatherScatterMode.PROMISE_IN_BOUNDS)


def _sc_reduce_body(contribs_hbm, lengths_hbm, out_hbm, scratch_hbm,
                    cv, lv, ov, gv, av):
    s = lax.axis_index("s")
    b = s
    lane = lax.iota(jnp.int32, 16)

    @pl.when(s < _B)
    def _work():
        pltpu.sync_copy(contribs_hbm.at[pl.ds(b * _M, _M)], cv)
        pltpu.sync_copy(lengths_hbm, lv)
        l_vec = _lane_shuffle(lv[...], jnp.full((16,), b, jnp.int32))

        def body(r, acc):
            for k in range(8):
                jl = pl.multiple_of(r * 128, 128) + k * 16
                v = cv[pl.ds(jl, 16)]
                m = (jl + lane) < l_vec
                acc = acc + jnp.where(m, v, 0.0)
            return acc

        v = lax.fori_loop(0, _M // 128, body, jnp.zeros((16,), jnp.float32))
        for shift in (8, 4, 2, 1):
            v = v + _lane_shuffle(v, lane ^ shift)
        ov[...] = v
        pltpu.sync_copy(ov, scratch_hbm.at[b])

    plsc.subcore_barrier()

    @pl.when(s == 0)
    def _assemble():
        pltpu.sync_copy(scratch_hbm, av)
        res = jnp.zeros((16,), jnp.float32)
        for bb in range(_B):
            res = res + jnp.where(lane == bb, av[bb], 0.0)
        gv[...] = res
        pltpu.sync_copy(gv, out_hbm)


def _sc_segment_sum(contribs, lengths):
    mesh = plsc.VectorSubcoreMesh(core_axis_name="c", subcore_axis_name="s",
                                  num_cores=1)
    f = pl.kernel(
        _sc_reduce_body,
        out_type=(jax.ShapeDtypeStruct((_B,), jnp.float32),
                  jax.ShapeDtypeStruct((_B, 16), jnp.float32)),
        mesh=mesh,
        scratch_types=[
            pltpu.VMEM((_M,), jnp.float32),
            pltpu.VMEM((_B,), jnp.int32),
            pltpu.VMEM((16,), jnp.float32),
            pltpu.VMEM((16,), jnp.float32),
            pltpu.VMEM((_B, 16), jnp.float32),
        ],
    )
    return f(contribs, lengths)[0]


def kernel(inputs, masks, sequence_lengths, W1, b1, W2, b2):
    xt = inputs.transpose(0, 2, 1)
    contribs = _subnet_contribs(
        xt, W1.T, b1.reshape(1, _H), W2.reshape(1, _H), b2.reshape(1, 1))
    return _sc_segment_sum(contribs, sequence_lengths)


# speedup vs baseline: 1.0230x; 1.0230x over previous
import jax
import jax.numpy as jnp
from jax import lax
from jax.experimental import pallas as pl
from jax.experimental.pallas import tpu as pltpu
from jax.experimental.pallas import tpu_sc as plsc

_B, _M, _F, _H = 16, 4096, 64, 16


_BI = 4
_NSTEP = _B // _BI


def _subnet_body(x_hbm, w1t_ref, b1r_ref, w2t_ref, b2_ref, out_ref,
                 xbuf, sems):
    i = pl.program_id(0)
    nbuf = 3

    @pl.when(i == 0)
    def _prime():
        for k in range(nbuf - 1):
            pltpu.make_async_copy(x_hbm.at[pl.ds(k * _BI, _BI)],
                                  xbuf.at[k], sems.at[k]).start()

    @pl.when(i + nbuf - 1 < _NSTEP)
    def _next():
        j = i + nbuf - 1
        pltpu.make_async_copy(x_hbm.at[pl.ds(j * _BI, _BI)],
                              xbuf.at[j % nbuf], sems.at[j % nbuf]).start()

    pltpu.make_async_copy(x_hbm.at[pl.ds(i * _BI, _BI)],
                          xbuf.at[i % nbuf], sems.at[i % nbuf]).wait()
    b1c = b1r_ref[...].reshape(_H, 1)
    for b in range(_BI):
        x = xbuf[i % nbuf, b]
        zt = lax.dot_general(w1t_ref[...], x, (((1,), (0,)), ((), ())),
                             preferred_element_type=jnp.float32)
        ht = jnp.maximum(zt + b1c, 0.0)
        c = lax.dot_general(w2t_ref[...], ht, (((1,), (0,)), ((), ())),
                            preferred_element_type=jnp.float32)
        out_ref[pl.ds(b * _M, _M)] = (c + b2_ref[0, 0]).reshape(_M)


def _subnet_contribs(xt, w1t, b1r, w2t, b2r):
    return pl.pallas_call(
        _subnet_body,
        grid=(_NSTEP,),
        in_specs=[
            pl.BlockSpec(memory_space=pl.ANY),
            pl.BlockSpec((_H, _F), lambda i: (0, 0)),
            pl.BlockSpec((1, _H), lambda i: (0, 0)),
            pl.BlockSpec((1, _H), lambda i: (0, 0)),
            pl.BlockSpec((1, 1), lambda i: (0, 0)),
        ],
        out_specs=pl.BlockSpec((_BI * _M,), lambda i: (i,)),
        out_shape=jax.ShapeDtypeStruct((_B * _M,), jnp.float32),
        scratch_shapes=[
            pltpu.VMEM((3, _BI, _F, _M), jnp.float32),
            pltpu.SemaphoreType.DMA((3,)),
        ],
    )(xt, w1t, b1r, w2t, b2r)


def _lane_shuffle(v, idx):
    return lax.gather(
        v, idx[:, None],
        lax.GatherDimensionNumbers(offset_dims=(), collapsed_slice_dims=(0,),
                                   start_index_map=(0,)),
        slice_sizes=(1,), mode=lax.GatherScatterMode.PROMISE_IN_BOUNDS)


def _sc_reduce_body(contribs_hbm, lengths_hbm, out_hbm, scratch_hbm,
                    cv, lv, ov, gv, av):
    s = lax.axis_index("s")
    b = s
    lane = lax.iota(jnp.int32, 16)

    @pl.when(s < _B)
    def _work():
        pltpu.sync_copy(contribs_hbm.at[pl.ds(b * _M, _M)], cv)
        pltpu.sync_copy(lengths_hbm, lv)
        l_vec = _lane_shuffle(lv[...], jnp.full((16,), b, jnp.int32))

        def body(r, acc):
            for k in range(8):
                jl = pl.multiple_of(r * 128, 128) + k * 16
                v = cv[pl.ds(jl, 16)]
                m = (jl + lane) < l_vec
                acc = acc + jnp.where(m, v, 0.0)
            return acc

        v = lax.fori_loop(0, _M // 128, body, jnp.zeros((16,), jnp.float32))
        for shift in (8, 4, 2, 1):
            v = v + _lane_shuffle(v, lane ^ shift)
        ov[...] = v
        pltpu.sync_copy(ov, scratch_hbm.at[b])

    plsc.subcore_barrier()

    @pl.when(s == 0)
    def _assemble():
        pltpu.sync_copy(scratch_hbm, av)
        res = jnp.zeros((16,), jnp.float32)
        for bb in range(_B):
            res = res + jnp.where(lane == bb, av[bb], 0.0)
        gv[...] = res
        pltpu.sync_copy(gv, out_hbm)


def _sc_segment_sum(contribs, lengths):
    mesh = plsc.VectorSubcoreMesh(core_axis_name="c", subcore_axis_name="s",
                                  num_cores=1)
    f = pl.kernel(
        _sc_reduce_body,
        out_type=(jax.ShapeDtypeStruct((_B,), jnp.float32),
                  jax.ShapeDtypeStruct((_B, 16), jnp.float32)),
        mesh=mesh,
        scratch_types=[
            pltpu.VMEM((_M,), jnp.float32),
            pltpu.VMEM((_B,), jnp.int32),
            pltpu.VMEM((16,), jnp.float32),
            pltpu.VMEM((16,), jnp.float32),
            pltpu.VMEM((_B, 16), jnp.float32),
        ],
    )
    return f(contribs, lengths)[0]


def kernel(inputs, masks, sequence_lengths, W1, b1, W2, b2):
    xt = inputs.transpose(0, 2, 1)
    contribs = _subnet_contribs(
        xt, W1.T, b1.reshape(1, _H), W2.reshape(1, _H), b2.reshape(1, 1))
    return _sc_segment_sum(contribs, sequence_lengths)
